# Initial kernel scaffold; baseline (speedup 1.0000x reference)
import jax
import jax.numpy as jnp
from jax.experimental import pallas as pl


def kernel(indices, table):
    raise NotImplementedError("write your pallas kernel here")


# speedup vs baseline: 1.1017x; 1.1017x over previous
import functools

import jax
import jax.numpy as jnp
from jax import lax
from jax.experimental import pallas as pl
from jax.experimental.pallas import tpu as pltpu
from jax.experimental.pallas import tpu_sc as plsc

_NUM_CORES = 2
_NUM_SUBCORES = 16
_NUM_WORKERS = _NUM_CORES * _NUM_SUBCORES

_CHUNK = 128
_GROUP = 8


def _gather_kernel(n_idx, emb, table_hbm, idx_hbm, out_hbm, idx_v, rows_v, sem):
    per_w = n_idx // _NUM_WORKERS
    n_chunks = per_w // _CHUNK
    n_outer = n_chunks // _GROUP
    wid = lax.axis_index("s") * _NUM_CORES + lax.axis_index("c")

    pltpu.sync_copy(idx_hbm.at[wid], idx_v)

    row_base = wid * per_w

    def body(g, carry):
        copies = []
        for j in range(_GROUP):
            c = g * _GROUP + j
            cp = pltpu.async_copy(
                table_hbm.at[idx_v.at[c]],
                rows_v.at[pl.ds(j * _CHUNK, _CHUNK)],
                sem,
            )
            copies.append(cp)
        for cp in copies:
            cp.wait()
        pltpu.sync_copy(
            rows_v,
            out_hbm.at[pl.ds(row_base + g * (_GROUP * _CHUNK), _GROUP * _CHUNK)],
        )
        return carry

    lax.fori_loop(0, n_outer, body, 0)


def kernel(indices, table):
    b, l = indices.shape
    n_idx = b * l
    emb = table.shape[1]
    per_w = n_idx // _NUM_WORKERS
    n_chunks = per_w // _CHUNK

    idx_flat = indices.reshape(_NUM_WORKERS, n_chunks, _CHUNK).astype(jnp.int32)

    mesh = plsc.VectorSubcoreMesh(core_axis_name="c", subcore_axis_name="s")
    out = pl.kernel(
        functools.partial(_gather_kernel, n_idx, emb),
        mesh=mesh,
        out_type=jax.ShapeDtypeStruct((n_idx, emb), jnp.float32),
        scratch_types=[
            pltpu.VMEM((n_chunks, _CHUNK), jnp.int32),
            pltpu.VMEM((_GROUP * _CHUNK, emb), jnp.float32),
            pltpu.SemaphoreType.DMA,
        ],
        compiler_params=pltpu.CompilerParams(use_tc_tiling_on_sc=False),
    )(table, idx_flat)
    return out.reshape(b, l, emb)


# speedup vs baseline: 1.1124x; 1.0097x over previous
import functools

import jax
import jax.numpy as jnp
from jax import lax
from jax.experimental import pallas as pl
from jax.experimental.pallas import tpu as pltpu
from jax.experimental.pallas import tpu_sc as plsc

_NUM_CORES = 2
_NUM_SUBCORES = 16
_NUM_WORKERS = _NUM_CORES * _NUM_SUBCORES

_CHUNK = 128
_GROUP = 10


def _gather_kernel(n_idx, emb, table_hbm, idx_hbm, out_hbm,
                   idx_v, buf0, buf1, sem0, sem1):
    per_w = n_idx // _NUM_WORKERS
    n_chunks = per_w // _CHUNK
    n_groups = n_chunks // _GROUP
    rows_per_group = _GROUP * _CHUNK
    wid = lax.axis_index("s") * _NUM_CORES + lax.axis_index("c")

    pltpu.sync_copy(idx_hbm.at[wid], idx_v)

    row_base = wid * per_w

    def fire(buf, sem, g):
        for j in range(_GROUP):
            pltpu.async_copy(
                table_hbm.at[idx_v.at[g * _GROUP + j]],
                buf.at[pl.ds(j * _CHUNK, _CHUNK)],
                sem,
            )

    def drain(buf, sem):
        pltpu.make_async_copy(
            table_hbm.at[pl.ds(0, rows_per_group)], buf, sem).wait()

    def writeback(buf, g):
        pltpu.sync_copy(
            buf,
            out_hbm.at[pl.ds(row_base + g * rows_per_group, rows_per_group)],
        )

    fire(buf0, sem0, 0)
    fire(buf1, sem1, 1)

    def body(t, carry):
        g0 = 2 * t
        g1 = g0 + 1

        drain(buf0, sem0)
        writeback(buf0, g0)

        @pl.when(g0 + 2 < n_groups)
        def _():
            fire(buf0, sem0, g0 + 2)

        drain(buf1, sem1)
        writeback(buf1, g1)

        @pl.when(g1 + 2 < n_groups)
        def _():
            fire(buf1, sem1, g1 + 2)

        return carry

    lax.fori_loop(0, n_groups // 2, body, 0)


def kernel(indices, table):
    b, l = indices.shape
    n_idx = b * l
    emb = table.shape[1]
    per_w = n_idx // _NUM_WORKERS
    n_chunks = per_w // _CHUNK

    idx_flat = indices.reshape(_NUM_WORKERS, n_chunks, _CHUNK).astype(jnp.int32)

    mesh = plsc.VectorSubcoreMesh(core_axis_name="c", subcore_axis_name="s")
    out = pl.kernel(
        functools.partial(_gather_kernel, n_idx, emb),
        mesh=mesh,
        out_type=jax.ShapeDtypeStruct((n_idx, emb), jnp.float32),
        scratch_types=[
            pltpu.VMEM((n_chunks, _CHUNK), jnp.int32),
            pltpu.VMEM((_GROUP * _CHUNK, emb), jnp.float32),
            pltpu.VMEM((_GROUP * _CHUNK, emb), jnp.float32),
            pltpu.SemaphoreType.DMA,
            pltpu.SemaphoreType.DMA,
        ],
        compiler_params=pltpu.CompilerParams(use_tc_tiling_on_sc=False),
    )(table, idx_flat)
    return out.reshape(b, l, emb)


# speedup vs baseline: 1.8042x; 1.6218x over previous
import functools

import jax
import jax.numpy as jnp
from jax import lax
from jax.experimental import pallas as pl
from jax.experimental.pallas import tpu as pltpu
from jax.experimental.pallas import tpu_sc as plsc

_NUM_CORES = 2
_NUM_SUBCORES = 16
_NUM_WORKERS = _NUM_CORES * _NUM_SUBCORES

_GROUP = 16


def _gather_kernel(batch, length, emb, table_hbm, idx_hbm, out_hbm,
                   idx_v, buf0, buf1, sem0, sem1):
    rows_w = batch // _NUM_WORKERS
    n_groups = rows_w // _GROUP
    wid = lax.axis_index("s") * _NUM_CORES + lax.axis_index("c")
    row_base = wid * rows_w

    pltpu.sync_copy(idx_hbm.at[pl.ds(row_base, rows_w)], idx_v)

    def fire(buf, sem, g):
        for j in range(_GROUP):
            pltpu.async_copy(
                table_hbm.at[idx_v.at[g * _GROUP + j]],
                buf.at[j],
                sem,
            )

    def drain(buf, sem):
        pltpu.make_async_copy(out_hbm.at[pl.ds(0, _GROUP)], buf, sem).wait()

    def writeback(buf, g):
        pltpu.sync_copy(
            buf, out_hbm.at[pl.ds(row_base + g * _GROUP, _GROUP)])

    fire(buf0, sem0, 0)
    fire(buf1, sem1, 1)

    def body(t, carry):
        g0 = 2 * t
        g1 = g0 + 1

        drain(buf0, sem0)
        writeback(buf0, g0)

        @pl.when(g0 + 2 < n_groups)
        def _():
            fire(buf0, sem0, g0 + 2)

        drain(buf1, sem1)
        writeback(buf1, g1)

        @pl.when(g1 + 2 < n_groups)
        def _():
            fire(buf1, sem1, g1 + 2)

        return carry

    lax.fori_loop(0, n_groups // 2, body, 0)


def kernel(indices, table):
    batch, length = indices.shape
    emb = table.shape[1]
    rows_w = batch // _NUM_WORKERS

    idx = indices.astype(jnp.int32)

    mesh = plsc.VectorSubcoreMesh(core_axis_name="c", subcore_axis_name="s")
    out = pl.kernel(
        functools.partial(_gather_kernel, batch, length, emb),
        mesh=mesh,
        out_type=jax.ShapeDtypeStruct((batch, length, emb), jnp.float32),
        scratch_types=[
            pltpu.VMEM((rows_w, length), jnp.int32),
            pltpu.VMEM((_GROUP, length, emb), jnp.float32),
            pltpu.VMEM((_GROUP, length, emb), jnp.float32),
            pltpu.SemaphoreType.DMA,
            pltpu.SemaphoreType.DMA,
        ],
        compiler_params=pltpu.CompilerParams(use_tc_tiling_on_sc=False),
    )(table, idx)
    return out


# speedup vs baseline: 1.9748x; 1.0946x over previous
import functools

import jax
import jax.numpy as jnp
from jax import lax
from jax.experimental import pallas as pl
from jax.experimental.pallas import tpu as pltpu
from jax.experimental.pallas import tpu_sc as plsc

_NUM_CORES = 2
_NUM_SUBCORES = 16
_NUM_WORKERS = _NUM_CORES * _NUM_SUBCORES

_GROUP = 4

_ROW = 128


def _gather_kernel(batch, length, emb, table_hbm, idx_hbm, out_hbm,
                   idx_v, buf0, buf1, sem0, sem1):
    rows_w = batch // _NUM_WORKERS
    n_groups = rows_w // _GROUP
    wid = lax.axis_index("s") * _NUM_CORES + lax.axis_index("c")
    row_base = wid * rows_w

    pltpu.sync_copy(idx_hbm.at[pl.ds(row_base, rows_w)], idx_v)

    def fire(buf, sem, g):
        for j in range(_GROUP):
            pltpu.async_copy(
                table_hbm.at[idx_v.at[g * _GROUP + j]],
                buf.at[j],
                sem,
            )

    def drain(buf, sem):
        pltpu.make_async_copy(out_hbm.at[pl.ds(0, _GROUP)], buf, sem).wait()

    def writeback(buf, g):
        pltpu.sync_copy(
            buf, out_hbm.at[pl.ds(row_base + g * _GROUP, _GROUP)])

    fire(buf0, sem0, 0)
    fire(buf1, sem1, 1)

    def body(t, carry):
        g0 = 2 * t
        g1 = g0 + 1

        drain(buf0, sem0)
        writeback(buf0, g0)

        @pl.when(g0 + 2 < n_groups)
        def _():
            fire(buf0, sem0, g0 + 2)

        drain(buf1, sem1)
        writeback(buf1, g1)

        @pl.when(g1 + 2 < n_groups)
        def _():
            fire(buf1, sem1, g1 + 2)

        return carry

    lax.fori_loop(0, n_groups // 2, body, 0)


def kernel(indices, table):
    batch, length = indices.shape
    emb = table.shape[1]
    rows_w = batch // _NUM_WORKERS

    idx = indices.astype(jnp.int32)
    table_pad = jnp.pad(table, ((0, 0), (0, _ROW - emb)))

    mesh = plsc.VectorSubcoreMesh(core_axis_name="c", subcore_axis_name="s")
    out = pl.kernel(
        functools.partial(_gather_kernel, batch, length, emb),
        mesh=mesh,
        out_type=jax.ShapeDtypeStruct((batch, length, _ROW), jnp.float32),
        scratch_types=[
            pltpu.VMEM((rows_w, length), jnp.int32),
            pltpu.VMEM((_GROUP, length, _ROW), jnp.float32),
            pltpu.VMEM((_GROUP, length, _ROW), jnp.float32),
            pltpu.SemaphoreType.DMA,
            pltpu.SemaphoreType.DMA,
        ],
        compiler_params=pltpu.CompilerParams(use_tc_tiling_on_sc=True),
    )(table_pad, idx)
    return out[..., :emb]
